# baseline (speedup 1.0000x reference)
import functools

import jax
import jax.numpy as jnp
from jax import lax
from jax.experimental import pallas as pl
from jax.experimental.pallas import tpu as pltpu, tpu_sc as plsc

_BS = 64
_NW = 32
_S = 512
_DB = 16
_RPW = _S // _NW


def _bias_kernel(row_x_ref, row_y_ref, col_x_ref, col_y_ref, tab_ref, out_ref):
    rx = row_x_ref[...][:, :, None]
    ry = row_y_ref[...][:, :, None]
    cx = col_x_ref[...][None, :, :]
    cy = col_y_ref[...][None, :, :]
    dist = jnp.abs(rx - cx) + jnp.abs(ry - cy)
    dist = jnp.minimum(dist, 8.0)
    acc = jnp.broadcast_to(tab_ref[0, :][None, None, :], dist.shape)
    for d in range(1, 9):
        acc = jnp.where(dist == float(d), tab_ref[d, :][None, None, :], acc)
    out_ref[...] = acc


def _tc_bias_plane(stab_xy, dist_emb, S, DB):
    xy = stab_xy.astype(jnp.float32)
    row_x = xy[:, 0:1]
    row_y = xy[:, 1:2]
    col_x = jnp.broadcast_to(xy[:, 0].reshape(S // 8, 8, 1), (S // 8, 8, DB)).reshape(S // 8, 8 * DB)
    col_y = jnp.broadcast_to(xy[:, 1].reshape(S // 8, 8, 1), (S // 8, 8, DB)).reshape(S // 8, 8 * DB)
    tab = jnp.tile(dist_emb, (1, 128 // DB))

    grid = (S // _BS,)
    plane = pl.pallas_call(
        _bias_kernel,
        grid=grid,
        in_specs=[
            pl.BlockSpec((_BS, 1), lambda i: (i, 0)),
            pl.BlockSpec((_BS, 1), lambda i: (i, 0)),
            pl.BlockSpec((S // 8, 128), lambda i: (0, 0)),
            pl.BlockSpec((S // 8, 128), lambda i: (0, 0)),
            pl.BlockSpec((9, 128), lambda i: (0, 0)),
        ],
        out_specs=pl.BlockSpec((_BS, S // 8, 128), lambda i: (i, 0, 0)),
        out_shape=jax.ShapeDtypeStruct((S, S // 8, 128), jnp.float32),
    )(row_x, row_y, col_x, col_y, tab)
    return plane


def _sc_broadcast_body(src_hbm, out_hbm, buf, wsem):
    b_sz = out_hbm.shape[0]
    chunk = buf.shape[1]
    src = src_hbm.reshape(_S * _S, _DB)
    out = out_hbm.reshape(b_sz * _S * _S, _DB)
    nch = _S // chunk
    nc = 2
    nbuf = buf.shape[0]
    wid = lax.axis_index("s") * nc + lax.axis_index("c")
    base = wid * _RPW

    pending = {}
    for i in range(_RPW * nch):
        slot = i % nbuf
        if i >= nbuf:
            for c in pending.pop(i - nbuf):
                c.wait()
        r = base + i // nch
        h = i % nch
        off = r * _S + h * chunk
        pltpu.sync_copy(src.at[pl.ds(off, chunk)], buf.at[slot])
        pending[i] = [
            pltpu.async_copy(buf.at[slot],
                             out.at[pl.ds(b * _S * _S + off, chunk)],
                             wsem.at[slot])
            for b in range(b_sz)
        ]
    for i in sorted(pending):
        for c in pending[i]:
            c.wait()


def _sc_broadcast(plane, B, S, DB):
    f = pl.kernel(
        _sc_broadcast_body,
        out_type=jax.ShapeDtypeStruct((B, S, S, DB), jnp.float32),
        mesh=plsc.VectorSubcoreMesh(core_axis_name="c", subcore_axis_name="s"),
        scratch_types=[
            pltpu.VMEM((4, 128, DB), jnp.float32),
            pltpu.SemaphoreType.DMA((4,)),
        ],
    )
    return f(plane.reshape(S, S, DB))


def kernel(stab_xy, syndrome, dist_emb, S):
    B = syndrome.shape[0]
    s_static = stab_xy.shape[0]
    DB = dist_emb.shape[1]
    plane = _tc_bias_plane(stab_xy, dist_emb, s_static, DB)
    return _sc_broadcast(plane, B, s_static, DB)


# speedup vs baseline: 5.3425x; 5.3425x over previous
import functools

import jax
import jax.numpy as jnp
from jax import lax
from jax.experimental import pallas as pl
from jax.experimental.pallas import tpu as pltpu, tpu_sc as plsc

_BS = 64
_NW = 32
_S = 512
_DB = 16
_RPW = _S // _NW


def _bias_kernel(row_x_ref, row_y_ref, col_x_ref, col_y_ref, tab_ref, out_ref):
    rx = row_x_ref[...][:, :, None]
    ry = row_y_ref[...][:, :, None]
    cx = col_x_ref[...][None, :, :]
    cy = col_y_ref[...][None, :, :]
    dist = jnp.abs(rx - cx) + jnp.abs(ry - cy)
    dist = jnp.minimum(dist, 8.0)
    acc = jnp.broadcast_to(tab_ref[0, :][None, None, :], dist.shape)
    for d in range(1, 9):
        acc = jnp.where(dist == float(d), tab_ref[d, :][None, None, :], acc)
    out_ref[...] = acc


def _tc_bias_plane(stab_xy, dist_emb, S, DB):
    xy = stab_xy.astype(jnp.float32)
    row_x = xy[:, 0:1]
    row_y = xy[:, 1:2]
    col_x = jnp.broadcast_to(xy[:, 0].reshape(S // 8, 8, 1), (S // 8, 8, DB)).reshape(S // 8, 8 * DB)
    col_y = jnp.broadcast_to(xy[:, 1].reshape(S // 8, 8, 1), (S // 8, 8, DB)).reshape(S // 8, 8 * DB)
    tab = jnp.tile(dist_emb, (1, 128 // DB))

    grid = (S // _BS,)
    plane = pl.pallas_call(
        _bias_kernel,
        grid=grid,
        in_specs=[
            pl.BlockSpec((_BS, 1), lambda i: (i, 0)),
            pl.BlockSpec((_BS, 1), lambda i: (i, 0)),
            pl.BlockSpec((S // 8, 128), lambda i: (0, 0)),
            pl.BlockSpec((S // 8, 128), lambda i: (0, 0)),
            pl.BlockSpec((9, 128), lambda i: (0, 0)),
        ],
        out_specs=pl.BlockSpec((_BS, S // 8, 128), lambda i: (i, 0, 0)),
        out_shape=jax.ShapeDtypeStruct((S, S // 8, 128), jnp.float32),
    )(row_x, row_y, col_x, col_y, tab)
    return plane


def _sc_broadcast_body(src_hbm, out_hbm, buf, wsem):
    b_sz = out_hbm.shape[0]
    src = src_hbm
    out = out_hbm
    nc = 2
    nbuf = buf.shape[0]
    wid = lax.axis_index("s") * nc + lax.axis_index("c")
    base = wid * _RPW

    pending = {}
    for i in range(_RPW):
        slot = i % nbuf
        if i >= nbuf:
            for c in pending.pop(i - nbuf):
                c.wait()
        r = base + i
        pltpu.sync_copy(src.at[r], buf.at[slot])
        pending[i] = [
            pltpu.async_copy(buf.at[slot], out.at[b, r], wsem.at[slot])
            for b in range(b_sz)
        ]
    for i in sorted(pending):
        for c in pending[i]:
            c.wait()


def _sc_broadcast(plane, B, S, DB):
    f = pl.kernel(
        _sc_broadcast_body,
        out_type=jax.ShapeDtypeStruct((B, S, S // 8, 128), jnp.float32),
        mesh=plsc.VectorSubcoreMesh(core_axis_name="c", subcore_axis_name="s"),
        scratch_types=[
            pltpu.VMEM((4, S // 8, 128), jnp.float32),
            pltpu.SemaphoreType.DMA((4,)),
        ],
    )
    return f(plane)


def _run_one(stab_xy, syndrome, dist_emb):
    B = syndrome.shape[0]
    s_static = stab_xy.shape[0]
    DB = dist_emb.shape[1]
    plane = _tc_bias_plane(stab_xy, dist_emb, s_static, DB)
    out = _sc_broadcast(plane, B, s_static, DB)
    return out.reshape(B, s_static, s_static, DB)


def kernel(stab_xy, syndrome, dist_emb, S):
    B = syndrome.shape[0]
    devs = jax.devices()
    nd = len(devs)
    while nd > 1 and B % nd != 0:
        nd -= 1
    if nd <= 1:
        return _run_one(stab_xy, syndrome, dist_emb)
    mesh = jax.sharding.Mesh(devs[:nd], ("b",))
    P = jax.sharding.PartitionSpec
    f = jax.shard_map(
        _run_one,
        mesh=mesh,
        in_specs=(P(), P("b"), P()),
        out_specs=P("b"),
        check_vma=False,
    )
    return f(stab_xy, syndrome, dist_emb)


# speedup vs baseline: 5.3580x; 1.0029x over previous
import jax
import jax.numpy as jnp
from jax import lax
from jax.experimental import pallas as pl
from jax.experimental.pallas import tpu as pltpu, tpu_sc as plsc

_BS = 64
_NW = 32
_S = 512
_DB = 16
_RPW = _S // _NW


def _bias_kernel(row_x_ref, row_y_ref, col_x_ref, col_y_ref, tab_ref, out_ref):
    rx = row_x_ref[...][:, :, None]
    ry = row_y_ref[...][:, :, None]
    cx = col_x_ref[...][None, :, :]
    cy = col_y_ref[...][None, :, :]
    dist = jnp.abs(rx - cx) + jnp.abs(ry - cy)
    dist = jnp.minimum(dist, 8.0)
    acc = jnp.broadcast_to(tab_ref[0, :][None, None, :], dist.shape)
    for d in range(1, 9):
        acc = jnp.where(dist == float(d), tab_ref[d, :][None, None, :], acc)
    out_ref[...] = acc


def _tc_bias_plane(stab_xy, dist_emb, S, DB):
    xy = stab_xy.astype(jnp.float32)
    row_x = xy[:, 0:1]
    row_y = xy[:, 1:2]
    col_x = jnp.broadcast_to(xy[:, 0].reshape(S // 8, 8, 1), (S // 8, 8, DB)).reshape(S // 8, 8 * DB)
    col_y = jnp.broadcast_to(xy[:, 1].reshape(S // 8, 8, 1), (S // 8, 8, DB)).reshape(S // 8, 8 * DB)
    tab = jnp.tile(dist_emb, (1, 128 // DB))

    grid = (S // _BS,)
    plane = pl.pallas_call(
        _bias_kernel,
        grid=grid,
        in_specs=[
            pl.BlockSpec((_BS, 1), lambda i: (i, 0)),
            pl.BlockSpec((_BS, 1), lambda i: (i, 0)),
            pl.BlockSpec((S // 8, 128), lambda i: (0, 0)),
            pl.BlockSpec((S // 8, 128), lambda i: (0, 0)),
            pl.BlockSpec((9, 128), lambda i: (0, 0)),
        ],
        out_specs=pl.BlockSpec((_BS, S // 8, 128), lambda i: (i, 0, 0)),
        out_shape=jax.ShapeDtypeStruct((S, S // 8, 128), jnp.float32),
    )(row_x, row_y, col_x, col_y, tab)
    return plane


def _sc_broadcast_body(src_hbm, out_hbm, buf, wsem):
    b_sz = out_hbm.shape[0]
    src = src_hbm
    out = out_hbm
    nc = 2
    nbuf = buf.shape[0]
    wid = lax.axis_index("s") * nc + lax.axis_index("c")
    base = wid * _RPW

    pending = {}
    for i in range(_RPW):
        slot = i % nbuf
        if i >= nbuf:
            for c in pending.pop(i - nbuf):
                c.wait()
        r = base + i
        pltpu.sync_copy(src.at[r], buf.at[slot])
        pending[i] = [
            pltpu.async_copy(buf.at[slot], out.at[b, r], wsem.at[slot])
            for b in range(b_sz)
        ]
    for i in sorted(pending):
        for c in pending[i]:
            c.wait()


def _sc_broadcast(plane, B, S, DB):
    f = pl.kernel(
        _sc_broadcast_body,
        out_type=jax.ShapeDtypeStruct((B, S, S // 8, 128), jnp.float32),
        mesh=plsc.VectorSubcoreMesh(core_axis_name="c", subcore_axis_name="s"),
        scratch_types=[
            pltpu.VMEM((4, S // 8, 128), jnp.float32),
            pltpu.SemaphoreType.DMA((4,)),
        ],
    )
    return f(plane)


def _run_one(stab_xy, syndrome, dist_emb):
    B = syndrome.shape[0]
    s_static = stab_xy.shape[0]
    DB = dist_emb.shape[1]
    plane = _tc_bias_plane(stab_xy, dist_emb, s_static, DB)
    out = _sc_broadcast(plane, B, s_static, DB)
    return out.reshape(B, s_static, s_static, DB)


def kernel(stab_xy, syndrome, dist_emb, S):
    B = syndrome.shape[0]
    devs = jax.devices()
    nd = len(devs)
    while nd > 1 and B % nd != 0:
        nd -= 1
    if nd <= 1:
        return _run_one(stab_xy, syndrome, dist_emb)
    mesh = jax.sharding.Mesh(devs[:nd], ("b",))
    P = jax.sharding.PartitionSpec
    f = jax.shard_map(
        _run_one,
        mesh=mesh,
        in_specs=(P(), P("b"), P()),
        out_specs=P("b"),
        check_vma=False,
    )
    return f(stab_xy, syndrome, dist_emb)
